# baseline (speedup 1.0000x reference)
import functools

import jax
import jax.numpy as jnp
from jax import lax
from jax.experimental import pallas as pl
from jax.experimental.pallas import tpu as pltpu
from jax.experimental.pallas import tpu_sc as plsc

N_NODES = 10000
D_FEAT = 128
D_OUT = 128

NC = 2
NS = 16
NW = NC * NS
L = 16
E_CHUNK = 128
STAGE_CH = 16
N_PAD = 10240
BLK = 512


@functools.lru_cache(maxsize=None)
def _sc_aggregate(n_edges_pad):
    nch = n_edges_pad // (NW * E_CHUNK)
    wb_rows = N_PAD // NS

    mesh = plsc.VectorSubcoreMesh(core_axis_name="c", subcore_axis_name="s")

    @functools.partial(
        pl.kernel,
        mesh=mesh,
        out_type=[
            jax.ShapeDtypeStruct((NC, N_PAD, D_FEAT), jnp.float32),
            jax.ShapeDtypeStruct((NW, N_PAD), jnp.float32),
        ],
        compiler_params=pltpu.CompilerParams(needs_layout_passes=False),
    scratch_types=[
            pltpu.VMEM((nch, E_CHUNK), jnp.int32),
            pltpu.VMEM((nch, E_CHUNK), jnp.int32),
            pltpu.VMEM((E_CHUNK, D_FEAT), jnp.float32),
            pltpu.VMEM((N_PAD,), jnp.float32),
            pltpu.VMEM_SHARED((N_PAD, D_FEAT), jnp.float32),
            pltpu.SemaphoreType.DMA,
        ],
    )
    def sc_agg(hs_hbm, src_hbm, dst_hbm, z_feat_hbm, z_deg_hbm,
               out_sum, out_deg,
               src_v, dst_v, rows_v, hist_v, acc_sh, sem):
        cid = lax.axis_index("c")
        sid = lax.axis_index("s")
        wid = cid * NS + sid

        pltpu.sync_copy(src_hbm.at[wid], src_v)
        pltpu.sync_copy(dst_hbm.at[wid], dst_v)
        pltpu.sync_copy(z_deg_hbm, hist_v)

        r0 = sid * wb_rows
        pltpu.sync_copy(z_feat_hbm.at[pl.ds(r0, wb_rows)],
                        acc_sh.at[pl.ds(r0, wb_rows)])
        plsc.subcore_barrier()

        ones16 = jnp.full((L,), 1.0, jnp.float32)

        def body(j, carry):
            cp = pltpu.async_copy(hs_hbm.at[src_v.at[j]], rows_v, sem)
            for k in range(E_CHUNK // L):
                dvec = dst_v[j, pl.ds(k * L, L)]
                plsc.addupdate_scatter(hist_v, [dvec], ones16)
            cp.wait()
            pltpu.sync_copy(rows_v, acc_sh.at[dst_v.at[j]], add=True)
            return carry

        lax.fori_loop(0, nch, body, 0)

        pltpu.sync_copy(hist_v, out_deg.at[wid])
        plsc.subcore_barrier()
        pltpu.sync_copy(acc_sh.at[pl.ds(r0, wb_rows)],
                        out_sum.at[cid, pl.ds(r0, wb_rows)])

    return sc_agg


def _tc_body(hd_ref, p_ref, dg_ref, w1_ref, w2_ref, b_ref, out_ref):
    s = p_ref[0] + p_ref[1]
    deg = jnp.sum(dg_ref[...], axis=0)[:, None]
    h_neigh = jnp.where(deg > 0.0, s / jnp.maximum(deg, 1.0), 0.0)
    out_ref[...] = (
        jnp.dot(hd_ref[...], w1_ref[...], preferred_element_type=jnp.float32)
        + jnp.dot(h_neigh, w2_ref[...], preferred_element_type=jnp.float32)
        + b_ref[...]
    )


def kernel(h_s, h_d, edge_index, W, b):
    src = edge_index[0].astype(jnp.int32)
    dst = edge_index[1].astype(jnp.int32)
    e = src.shape[0]
    tile_quant = NW * E_CHUNK * STAGE_CH
    e_pad = tile_quant * ((e + tile_quant - 1) // tile_quant)
    src_p = jnp.concatenate(
        [src, jnp.zeros((e_pad - e,), jnp.int32)]).reshape(NW, -1, E_CHUNK)
    dst_p = jnp.concatenate(
        [dst, jnp.full((e_pad - e,), N_NODES, jnp.int32)]).reshape(NW, -1, E_CHUNK)
    z_feat = jnp.zeros((N_PAD, D_FEAT), jnp.float32)
    z_deg = jnp.zeros((N_PAD,), jnp.float32)

    p, dg = _sc_aggregate(e_pad)(h_s, src_p, dst_p, z_feat, z_deg)

    w1t = W[:, :D_FEAT].T
    w2t = W[:, D_FEAT:].T
    b2 = b.reshape(1, D_OUT)

    n = h_d.shape[0]
    out = pl.pallas_call(
        _tc_body,
        grid=(N_PAD // BLK,),
        in_specs=[
            pl.BlockSpec((BLK, D_FEAT), lambda i: (i, 0)),
            pl.BlockSpec((NC, BLK, D_FEAT), lambda i: (0, i, 0)),
            pl.BlockSpec((NW, BLK), lambda i: (0, i)),
            pl.BlockSpec((D_FEAT, D_OUT), lambda i: (0, 0)),
            pl.BlockSpec((D_FEAT, D_OUT), lambda i: (0, 0)),
            pl.BlockSpec((1, D_OUT), lambda i: (0, 0)),
        ],
        out_specs=pl.BlockSpec((BLK, D_OUT), lambda i: (i, 0)),
        out_shape=jax.ShapeDtypeStruct((n, D_OUT), jnp.float32),
    )(h_d, p, dg, w1t, w2t, b2)
    return out


# speedup vs baseline: 2.8062x; 2.8062x over previous
import functools

import jax
import jax.numpy as jnp
from jax import lax
from jax.experimental import pallas as pl
from jax.experimental.pallas import tpu as pltpu
from jax.experimental.pallas import tpu_sc as plsc

N_NODES = 10000
D_FEAT = 128
D_OUT = 128

NC = 2
NS = 16
NW = NC * NS
L = 16
E_CHUNK = 128
STAGE_CH = 16
N_PAD = 10240
BLK = 512


@functools.lru_cache(maxsize=None)
def _sc_aggregate(n_edges_pad):
    nch = n_edges_pad // (NW * E_CHUNK)
    wb_rows = N_PAD // NS

    mesh = plsc.VectorSubcoreMesh(core_axis_name="c", subcore_axis_name="s")

    @functools.partial(
        pl.kernel,
        mesh=mesh,
        out_type=[
            jax.ShapeDtypeStruct((NC, N_PAD, D_FEAT), jnp.float32),
            jax.ShapeDtypeStruct((NW, N_PAD), jnp.float32),
        ],
        compiler_params=pltpu.CompilerParams(needs_layout_passes=False),
    scratch_types=[
            pltpu.VMEM((nch, E_CHUNK), jnp.int32),
            pltpu.VMEM((nch, E_CHUNK), jnp.int32),
            pltpu.VMEM((E_CHUNK, D_FEAT), jnp.float32),
            pltpu.VMEM((N_PAD,), jnp.float32),
            pltpu.VMEM_SHARED((N_PAD, D_FEAT), jnp.float32),
            pltpu.SemaphoreType.DMA,
        ],
    )
    def sc_agg(hs_hbm, src_hbm, dst_hbm, z_feat_hbm, z_deg_hbm,
               out_sum, out_deg,
               src_v, dst_v, rows_v, hist_v, acc_sh, sem):
        cid = lax.axis_index("c")
        sid = lax.axis_index("s")
        wid = cid * NS + sid

        pltpu.sync_copy(src_hbm.at[wid], src_v)
        pltpu.sync_copy(dst_hbm.at[wid], dst_v)
        pltpu.sync_copy(z_deg_hbm, hist_v)

        r0 = sid * wb_rows
        pltpu.sync_copy(z_feat_hbm.at[pl.ds(r0, wb_rows)],
                        acc_sh.at[pl.ds(r0, wb_rows)])
        plsc.subcore_barrier()

        ones16 = jnp.full((L,), 1.0, jnp.float32)

        def body(j, carry):
            cp = pltpu.async_copy(hs_hbm.at[src_v.at[j]], rows_v, sem)
            for k in range(E_CHUNK // L):
                dvec = dst_v[j, pl.ds(k * L, L)]
                plsc.addupdate_scatter(hist_v, [dvec], ones16)
            cp.wait()
            pltpu.sync_copy(rows_v, acc_sh.at[dst_v.at[j]], add=True)
            return carry

        lax.fori_loop(0, nch, body, 0)

        pltpu.sync_copy(hist_v, out_deg.at[wid])
        plsc.subcore_barrier()
        pltpu.sync_copy(acc_sh.at[pl.ds(r0, wb_rows)],
                        out_sum.at[cid, pl.ds(r0, wb_rows)])

    return sc_agg


def _tc_body(hd_ref, p_ref, dg_ref, w1_ref, w2_ref, b_ref, out_ref):
    s = p_ref[0] + p_ref[1]
    deg = jnp.sum(dg_ref[...], axis=0)[:, None]
    h_neigh = jnp.where(deg > 0.0, s / jnp.maximum(deg, 1.0), 0.0)
    out_ref[...] = (
        jnp.dot(hd_ref[...], w1_ref[...], preferred_element_type=jnp.float32)
        + jnp.dot(h_neigh, w2_ref[...], preferred_element_type=jnp.float32)
        + b_ref[...]
    )


def kernel(h_s, h_d, edge_index, W, b):
    src = edge_index[0].astype(jnp.int32)
    dst = edge_index[1].astype(jnp.int32)
    e = src.shape[0]
    tile_e = E_CHUNK * STAGE_CH * (
        (e + NW * E_CHUNK * STAGE_CH - 1) // (NW * E_CHUNK * STAGE_CH))
    e_pt = e // NW
    pad_pt = tile_e - e_pt
    dummy_src = jnp.broadcast_to(
        (jnp.arange(pad_pt, dtype=jnp.int32) % N_NODES)[None], (NW, pad_pt))
    dummy_dst = jnp.broadcast_to(
        (N_NODES + jnp.arange(pad_pt, dtype=jnp.int32) % (N_PAD - N_NODES))[None],
        (NW, pad_pt))
    src_p = jnp.concatenate(
        [src.reshape(NW, e_pt), dummy_src], axis=1).reshape(NW, -1, E_CHUNK)
    dst_p = jnp.concatenate(
        [dst.reshape(NW, e_pt), dummy_dst], axis=1).reshape(NW, -1, E_CHUNK)
    e_pad = NW * tile_e
    z_feat = jnp.zeros((N_PAD, D_FEAT), jnp.float32)
    z_deg = jnp.zeros((N_PAD,), jnp.float32)

    p, dg = _sc_aggregate(e_pad)(h_s, src_p, dst_p, z_feat, z_deg)

    w1t = W[:, :D_FEAT].T
    w2t = W[:, D_FEAT:].T
    b2 = b.reshape(1, D_OUT)

    n = h_d.shape[0]
    out = pl.pallas_call(
        _tc_body,
        grid=(N_PAD // BLK,),
        in_specs=[
            pl.BlockSpec((BLK, D_FEAT), lambda i: (i, 0)),
            pl.BlockSpec((NC, BLK, D_FEAT), lambda i: (0, i, 0)),
            pl.BlockSpec((NW, BLK), lambda i: (0, i)),
            pl.BlockSpec((D_FEAT, D_OUT), lambda i: (0, 0)),
            pl.BlockSpec((D_FEAT, D_OUT), lambda i: (0, 0)),
            pl.BlockSpec((1, D_OUT), lambda i: (0, 0)),
        ],
        out_specs=pl.BlockSpec((BLK, D_OUT), lambda i: (i, 0)),
        out_shape=jax.ShapeDtypeStruct((n, D_OUT), jnp.float32),
    )(h_d, p, dg, w1t, w2t, b2)
    return out


# speedup vs baseline: 3.7137x; 1.3234x over previous
import functools

import jax
import jax.numpy as jnp
from jax import lax
from jax.experimental import pallas as pl
from jax.experimental.pallas import tpu as pltpu
from jax.experimental.pallas import tpu_sc as plsc

N_NODES = 10000
D_FEAT = 128
D_OUT = 128

NC = 2
NS = 16
NW = NC * NS
L = 16
E_CHUNK = 128
STAGE_CH = 16
N_PAD = 10240
BLK = 512


@functools.lru_cache(maxsize=None)
def _sc_aggregate(n_edges_pad):
    nch = n_edges_pad // (NW * E_CHUNK)
    wb_rows = N_PAD // NS

    mesh = plsc.VectorSubcoreMesh(core_axis_name="c", subcore_axis_name="s")

    @functools.partial(
        pl.kernel,
        mesh=mesh,
        out_type=[
            jax.ShapeDtypeStruct((NC, N_PAD, D_FEAT), jnp.float32),
            jax.ShapeDtypeStruct((NW, N_PAD), jnp.float32),
        ],
        compiler_params=pltpu.CompilerParams(needs_layout_passes=False),
    scratch_types=[
            pltpu.VMEM((STAGE_CH, E_CHUNK), jnp.int32),
            pltpu.VMEM((STAGE_CH, E_CHUNK), jnp.int32),
            pltpu.VMEM((E_CHUNK, D_FEAT), jnp.float32),
            pltpu.VMEM((E_CHUNK, D_FEAT), jnp.float32),
            pltpu.VMEM((N_PAD,), jnp.float32),
            pltpu.VMEM_SHARED((N_PAD, D_FEAT), jnp.float32),
            pltpu.SemaphoreType.DMA,
            pltpu.SemaphoreType.DMA,
            pltpu.SemaphoreType.DMA,
            pltpu.SemaphoreType.DMA,
        ],
    )
    def sc_agg(hs_hbm, src_hbm, dst_hbm, z_feat_hbm, z_deg_hbm,
               out_sum, out_deg,
               src_v, dst_v, rows0_v, rows1_v, hist_v, acc_sh,
               g0sem, g1sem, s0sem, s1sem):
        cid = lax.axis_index("c")
        sid = lax.axis_index("s")
        wid = cid * NS + sid

        pltpu.sync_copy(z_deg_hbm, hist_v)

        r0 = sid * wb_rows
        pltpu.sync_copy(z_feat_hbm.at[pl.ds(r0, wb_rows)],
                        acc_sh.at[pl.ds(r0, wb_rows)])
        plsc.subcore_barrier()

        ones16 = jnp.full((L,), 1.0, jnp.float32)
        npair = STAGE_CH // 2
        nstage = nch // STAGE_CH

        def start_gather(j, buf, sem):
            pltpu.make_async_copy(hs_hbm.at[src_v.at[j]], buf, sem).start()

        def wait_gather(buf, sem):
            pltpu.make_async_copy(hs_hbm.at[src_v.at[0]], buf, sem).wait()

        def start_scatter(j, buf, sem):
            pltpu.make_async_copy(buf, acc_sh.at[dst_v.at[j]], sem).start(add=True)

        def wait_scatter(j, buf, sem):
            pltpu.make_async_copy(buf, acc_sh.at[dst_v.at[j]], sem).wait()

        def hist(j):
            for k in range(E_CHUNK // L):
                dvec = dst_v[j, pl.ds(k * L, L)]
                plsc.addupdate_scatter(hist_v, [dvec], ones16)

        def stage_body(st, carry):
            pltpu.sync_copy(src_hbm.at[wid, pl.ds(st * STAGE_CH, STAGE_CH)],
                            src_v)
            pltpu.sync_copy(dst_hbm.at[wid, pl.ds(st * STAGE_CH, STAGE_CH)],
                            dst_v)
            start_gather(0, rows0_v, g0sem)

            def body(g, carry2):
                j0 = 2 * g
                j1 = j0 + 1
                start_gather(j1, rows1_v, g1sem)
                wait_gather(rows0_v, g0sem)
                start_scatter(j0, rows0_v, s0sem)
                hist(j0)
                wait_scatter(j0, rows0_v, s0sem)

                @pl.when(g < npair - 1)
                def _():
                    start_gather(j0 + 2, rows0_v, g0sem)

                wait_gather(rows1_v, g1sem)
                start_scatter(j1, rows1_v, s1sem)
                hist(j1)
                wait_scatter(j1, rows1_v, s1sem)
                return carry2

            lax.fori_loop(0, npair, body, 0)
            return carry

        lax.fori_loop(0, nstage, stage_body, 0)

        pltpu.sync_copy(hist_v, out_deg.at[wid])
        plsc.subcore_barrier()
        pltpu.sync_copy(acc_sh.at[pl.ds(r0, wb_rows)],
                        out_sum.at[cid, pl.ds(r0, wb_rows)])

    return sc_agg


def _tc_body(hd_ref, p_ref, dg_ref, w1_ref, w2_ref, b_ref, out_ref):
    s = p_ref[0] + p_ref[1]
    deg = jnp.sum(dg_ref[...], axis=0)[:, None]
    h_neigh = jnp.where(deg > 0.0, s / jnp.maximum(deg, 1.0), 0.0)
    out_ref[...] = (
        jnp.dot(hd_ref[...], w1_ref[...], preferred_element_type=jnp.float32)
        + jnp.dot(h_neigh, w2_ref[...], preferred_element_type=jnp.float32)
        + b_ref[...]
    )


def kernel(h_s, h_d, edge_index, W, b):
    src = edge_index[0].astype(jnp.int32)
    dst = edge_index[1].astype(jnp.int32)
    e = src.shape[0]
    tile_e = E_CHUNK * STAGE_CH * (
        (e + NW * E_CHUNK * STAGE_CH - 1) // (NW * E_CHUNK * STAGE_CH))
    e_pt = e // NW
    pad_pt = tile_e - e_pt
    dummy_src = jnp.broadcast_to(
        (jnp.arange(pad_pt, dtype=jnp.int32) % N_NODES)[None], (NW, pad_pt))
    dummy_dst = jnp.broadcast_to(
        (N_NODES + jnp.arange(pad_pt, dtype=jnp.int32) % (N_PAD - N_NODES))[None],
        (NW, pad_pt))
    src_p = jnp.concatenate(
        [src.reshape(NW, e_pt), dummy_src], axis=1).reshape(NW, -1, E_CHUNK)
    dst_p = jnp.concatenate(
        [dst.reshape(NW, e_pt), dummy_dst], axis=1).reshape(NW, -1, E_CHUNK)
    e_pad = NW * tile_e
    z_feat = jnp.zeros((N_PAD, D_FEAT), jnp.float32)
    z_deg = jnp.zeros((N_PAD,), jnp.float32)

    p, dg = _sc_aggregate(e_pad)(h_s, src_p, dst_p, z_feat, z_deg)

    w1t = W[:, :D_FEAT].T
    w2t = W[:, D_FEAT:].T
    b2 = b.reshape(1, D_OUT)

    n = h_d.shape[0]
    out = pl.pallas_call(
        _tc_body,
        grid=(N_PAD // BLK,),
        in_specs=[
            pl.BlockSpec((BLK, D_FEAT), lambda i: (i, 0)),
            pl.BlockSpec((NC, BLK, D_FEAT), lambda i: (0, i, 0)),
            pl.BlockSpec((NW, BLK), lambda i: (0, i)),
            pl.BlockSpec((D_FEAT, D_OUT), lambda i: (0, 0)),
            pl.BlockSpec((D_FEAT, D_OUT), lambda i: (0, 0)),
            pl.BlockSpec((1, D_OUT), lambda i: (0, 0)),
        ],
        out_specs=pl.BlockSpec((BLK, D_OUT), lambda i: (i, 0)),
        out_shape=jax.ShapeDtypeStruct((n, D_OUT), jnp.float32),
    )(h_d, p, dg, w1t, w2t, b2)
    return out


# speedup vs baseline: 3.9414x; 1.0613x over previous
import functools

import jax
import jax.numpy as jnp
from jax import lax
from jax.experimental import pallas as pl
from jax.experimental.pallas import tpu as pltpu
from jax.experimental.pallas import tpu_sc as plsc

N_NODES = 10000
D_FEAT = 128
D_OUT = 128

NC = 2
NS = 16
NW = NC * NS
L = 16
E_CHUNK = 128
STAGE_CH = 16
N_PAD = 10240
BLK = 2048


@functools.lru_cache(maxsize=None)
def _sc_aggregate(n_edges_pad):
    nch = n_edges_pad // (NW * E_CHUNK)
    wb_rows = N_PAD // NS

    mesh = plsc.VectorSubcoreMesh(core_axis_name="c", subcore_axis_name="s")

    @functools.partial(
        pl.kernel,
        mesh=mesh,
        out_type=[
            jax.ShapeDtypeStruct((NC, N_PAD, D_FEAT), jnp.float32),
            jax.ShapeDtypeStruct((NW, N_PAD), jnp.float32),
        ],
        compiler_params=pltpu.CompilerParams(needs_layout_passes=False),
    scratch_types=[
            pltpu.VMEM((STAGE_CH, E_CHUNK), jnp.int32),
            pltpu.VMEM((STAGE_CH, E_CHUNK), jnp.int32),
            pltpu.VMEM((E_CHUNK, D_FEAT), jnp.float32),
            pltpu.VMEM((E_CHUNK, D_FEAT), jnp.float32),
            pltpu.VMEM((N_PAD,), jnp.float32),
            pltpu.VMEM_SHARED((N_PAD, D_FEAT), jnp.float32),
            pltpu.SemaphoreType.DMA,
            pltpu.SemaphoreType.DMA,
            pltpu.SemaphoreType.DMA,
            pltpu.SemaphoreType.DMA,
        ],
    )
    def sc_agg(hs_hbm, src_hbm, dst_hbm, z_feat_hbm, z_deg_hbm,
               out_sum, out_deg,
               src_v, dst_v, rows0_v, rows1_v, hist_v, acc_sh,
               g0sem, g1sem, s0sem, s1sem):
        cid = lax.axis_index("c")
        sid = lax.axis_index("s")
        wid = cid * NS + sid

        pltpu.sync_copy(z_deg_hbm, hist_v)

        r0 = sid * wb_rows
        pltpu.sync_copy(z_feat_hbm.at[pl.ds(r0, wb_rows)],
                        acc_sh.at[pl.ds(r0, wb_rows)])
        plsc.subcore_barrier()

        ones16 = jnp.full((L,), 1.0, jnp.float32)
        npair = STAGE_CH // 2
        nstage = nch // STAGE_CH

        def start_gather(j, buf, sem):
            pltpu.make_async_copy(hs_hbm.at[src_v.at[j]], buf, sem).start()

        def wait_gather(buf, sem):
            pltpu.make_async_copy(hs_hbm.at[src_v.at[0]], buf, sem).wait()

        def start_scatter(j, buf, sem):
            pltpu.make_async_copy(buf, acc_sh.at[dst_v.at[j]], sem).start(add=True)

        def wait_scatter(j, buf, sem):
            pltpu.make_async_copy(buf, acc_sh.at[dst_v.at[j]], sem).wait()

        def hist(j):
            for k in range(E_CHUNK // L):
                dvec = dst_v[j, pl.ds(k * L, L)]
                plsc.addupdate_scatter(hist_v, [dvec], ones16)

        def stage_body(st, carry):
            pltpu.sync_copy(src_hbm.at[wid, pl.ds(st * STAGE_CH, STAGE_CH)],
                            src_v)
            pltpu.sync_copy(dst_hbm.at[wid, pl.ds(st * STAGE_CH, STAGE_CH)],
                            dst_v)
            start_gather(0, rows0_v, g0sem)

            def body(g, carry2):
                j0 = 2 * g
                j1 = j0 + 1
                start_gather(j1, rows1_v, g1sem)
                wait_gather(rows0_v, g0sem)
                start_scatter(j0, rows0_v, s0sem)
                hist(j0)
                wait_scatter(j0, rows0_v, s0sem)

                @pl.when(g < npair - 1)
                def _():
                    start_gather(j0 + 2, rows0_v, g0sem)

                wait_gather(rows1_v, g1sem)
                start_scatter(j1, rows1_v, s1sem)
                hist(j1)
                wait_scatter(j1, rows1_v, s1sem)
                return carry2

            lax.fori_loop(0, npair, body, 0)
            return carry

        lax.fori_loop(0, nstage, stage_body, 0)

        pltpu.sync_copy(hist_v, out_deg.at[wid])
        plsc.subcore_barrier()
        pltpu.sync_copy(acc_sh.at[pl.ds(r0, wb_rows)],
                        out_sum.at[cid, pl.ds(r0, wb_rows)])

    return sc_agg


def _tc_body(hd_ref, p_ref, dg_ref, w1_ref, w2_ref, b_ref, out_ref):
    s = p_ref[0] + p_ref[1]
    deg = jnp.sum(dg_ref[...], axis=0)[:, None]
    h_neigh = jnp.where(deg > 0.0, s / jnp.maximum(deg, 1.0), 0.0)
    out_ref[...] = (
        jnp.dot(hd_ref[...], w1_ref[...], preferred_element_type=jnp.float32)
        + jnp.dot(h_neigh, w2_ref[...], preferred_element_type=jnp.float32)
        + b_ref[...]
    )


def kernel(h_s, h_d, edge_index, W, b):
    src = edge_index[0].astype(jnp.int32)
    dst = edge_index[1].astype(jnp.int32)
    e = src.shape[0]
    tile_e = E_CHUNK * STAGE_CH * (
        (e + NW * E_CHUNK * STAGE_CH - 1) // (NW * E_CHUNK * STAGE_CH))
    e_pt = e // NW
    pad_pt = tile_e - e_pt
    dummy_src = jnp.broadcast_to(
        (jnp.arange(pad_pt, dtype=jnp.int32) % N_NODES)[None], (NW, pad_pt))
    dummy_dst = jnp.broadcast_to(
        (N_NODES + jnp.arange(pad_pt, dtype=jnp.int32) % (N_PAD - N_NODES))[None],
        (NW, pad_pt))
    src_p = jnp.concatenate(
        [src.reshape(NW, e_pt), dummy_src], axis=1).reshape(NW, -1, E_CHUNK)
    dst_p = jnp.concatenate(
        [dst.reshape(NW, e_pt), dummy_dst], axis=1).reshape(NW, -1, E_CHUNK)
    e_pad = NW * tile_e
    z_feat = jnp.zeros((N_PAD, D_FEAT), jnp.float32)
    z_deg = jnp.zeros((N_PAD,), jnp.float32)

    p, dg = _sc_aggregate(e_pad)(h_s, src_p, dst_p, z_feat, z_deg)

    w1t = W[:, :D_FEAT].T
    w2t = W[:, D_FEAT:].T
    b2 = b.reshape(1, D_OUT)

    n = h_d.shape[0]
    out = pl.pallas_call(
        _tc_body,
        grid=(N_PAD // BLK,),
        in_specs=[
            pl.BlockSpec((BLK, D_FEAT), lambda i: (i, 0)),
            pl.BlockSpec((NC, BLK, D_FEAT), lambda i: (0, i, 0)),
            pl.BlockSpec((NW, BLK), lambda i: (0, i)),
            pl.BlockSpec((D_FEAT, D_OUT), lambda i: (0, 0)),
            pl.BlockSpec((D_FEAT, D_OUT), lambda i: (0, 0)),
            pl.BlockSpec((1, D_OUT), lambda i: (0, 0)),
        ],
        out_specs=pl.BlockSpec((BLK, D_OUT), lambda i: (i, 0)),
        out_shape=jax.ShapeDtypeStruct((n, D_OUT), jnp.float32),
    )(h_d, p, dg, w1t, w2t, b2)
    return out
